# baseline (speedup 1.0000x reference)
import jax
import jax.numpy as jnp
from jax.experimental import pallas as pl
from jax.experimental.pallas import tpu as pltpu

_RB = 8
_K = 64
_EPS = 1e-05


_NEG = -1e30
_NC = 784


def _topk_extract_kernel(x_ref, q_ref, t_ref, topv_ref, topi_ref, topq_ref,
                         xs_ref, g_ref, qg_ref, og_ref):
    for row in range(_RB):
        t = t_ref[row, 0]
        t = jnp.where(t < _EPS, 1.0, t)
        xs_ref[row] = x_ref[row] / t

    cm = jnp.max(xs_ref[...], axis=2)

    ncio = jax.lax.broadcasted_iota(jnp.int32, (_RB, _NC), 1)
    lanek = jax.lax.broadcasted_iota(jnp.int32, (_RB, _K), 1)

    def cbody(tt, carry):
        cmx, selc = carry
        m = jnp.max(cmx, axis=1, keepdims=True)
        cid = jnp.min(jnp.where(cmx == m, ncio, _NC), axis=1, keepdims=True)
        cmx = jnp.where(ncio == cid, -jnp.inf, cmx)
        selc = jnp.where(lanek == tt, cid, selc)
        return cmx, selc

    _, selc = jax.lax.fori_loop(
        0, _K, cbody, (cm, jnp.zeros((_RB, _K), jnp.int32)))

    cio2 = jax.lax.broadcasted_iota(jnp.int32, (_K, _NC), 1)
    lane2 = jax.lax.broadcasted_iota(jnp.int32, (_K, 128), 1)
    for row in range(_RB):
        selc_col = jnp.swapaxes(selc[row:row + 1, :], 0, 1)
        oh = (cio2 == selc_col).astype(jnp.float32)
        g_ref[row] = jax.lax.dot_general(
            oh, xs_ref[row], (((1,), (0,)), ((), ())),
            preferred_element_type=jnp.float32,
            precision=jax.lax.Precision.HIGHEST)
        qg_ref[row] = jax.lax.dot_general(
            oh, q_ref[row], (((1,), (0,)), ((), ())),
            preferred_element_type=jnp.float32,
            precision=jax.lax.Precision.HIGHEST)
        og_ref[row] = selc_col * 128 + lane2

    orig3 = og_ref[...]
    qg3 = qg_ref[...]
    lanek3 = jax.lax.broadcasted_iota(jnp.int32, (_RB, _K, 1), 1)

    def ebody(tt, carry):
        gx, topv, topi, topq = carry
        m = jnp.max(jnp.max(gx, axis=2, keepdims=True), axis=1,
                    keepdims=True)
        cand = jnp.where(gx == m, orig3, _NC * 128)
        idx = jnp.min(jnp.min(cand, axis=2, keepdims=True), axis=1,
                      keepdims=True)
        sel = orig3 == idx
        qv = jnp.sum(jnp.sum(jnp.where(sel, qg3, 0.0), axis=2,
                             keepdims=True), axis=1, keepdims=True)
        gx = jnp.where(sel, _NEG, gx)
        hit = lanek3 == tt
        topv = jnp.where(hit, m, topv)
        topi = jnp.where(hit, idx, topi)
        topq = jnp.where(hit, qv, topq)
        return gx, topv, topi, topq

    init = (g_ref[...], jnp.full((_RB, _K, 1), -jnp.inf, jnp.float32),
            jnp.zeros((_RB, _K, 1), jnp.int32),
            jnp.ones((_RB, _K, 1), jnp.float32))
    _, topv, topi, topq = jax.lax.fori_loop(0, _K, ebody, init)
    topv_ref[...] = topv
    topi_ref[...] = topi
    topq_ref[...] = topq


def _finish_kernel(topv_ref, topi_ref, topq_ref, temp_ref, k_ref, p_ref,
                   samp_ref, oidx_ref, olp_ref):
    v = topv_ref[...]
    topi = topi_ref[...]
    topq = topq_ref[...]
    temp = temp_ref[...]
    kk = k_ref[...]
    p = p_ref[...]
    b = v.shape[0]

    j = jax.lax.broadcasted_iota(jnp.int32, (b, _K), 1)
    rio = jax.lax.broadcasted_iota(jnp.int32, (_K, _K), 0)
    cio = jax.lax.broadcasted_iota(jnp.int32, (_K, _K), 1)
    m_lt = (rio < cio).astype(jnp.float32)

    kthv = jnp.max(jnp.where(j == jnp.maximum(kk, 1) - 1, v, -jnp.inf),
                   axis=1, keepdims=True)
    keep_k = v >= kthv
    mx = v[:, 0:1]
    e = jnp.exp(jnp.where(keep_k, v, -jnp.inf) - mx)
    z = jnp.sum(e, axis=1, keepdims=True)
    pr = e / z

    ntie_before = jnp.zeros((b, _K), jnp.float32)
    ntie_after = jnp.zeros((b, _K), jnp.float32)
    run = jnp.zeros((b, 1), jnp.float32)
    prev = jnp.full((b, 1), jnp.inf, jnp.float32)
    for t in range(_K):
        v_t = jnp.max(jnp.where(j == t, v, -jnp.inf), axis=1, keepdims=True)
        run = jnp.where(v_t == prev, run + 1.0, 0.0)
        ntie_before = jnp.where(j == t, run, ntie_before)
        prev = v_t
    run = jnp.zeros((b, 1), jnp.float32)
    nxt = jnp.full((b, 1), jnp.inf, jnp.float32)
    for t in range(_K - 1, -1, -1):
        v_t = jnp.max(jnp.where(j == t, v, -jnp.inf), axis=1, keepdims=True)
        run = jnp.where(v_t == nxt, run + 1.0, 0.0)
        ntie_after = jnp.where(j == t, run, ntie_after)
        nxt = v_t

    s_inc = jax.lax.dot_general(pr, (rio <= cio).astype(jnp.float32),
                                (((1,), (0,)), ((), ())),
                                preferred_element_type=jnp.float32,
                            precision=jax.lax.Precision.HIGHEST)
    sgt = (s_inc - pr) - pr * ntie_before
    s_prev = sgt + pr * ntie_after
    is_top0 = (v == mx) & (ntie_after == 0.0)
    keep_p = is_top0 | (s_prev < p)
    kept = keep_k & keep_p
    kf = kept.astype(jnp.float32)
    lnf = jnp.sum(kf, axis=1, keepdims=True)
    ln = lnf.astype(jnp.int32)

    xf = jnp.where(kept, v, -jnp.inf)
    e2 = jnp.exp(xf - mx)
    z2 = jnp.sum(e2, axis=1, keepdims=True)
    pr2 = e2 / z2
    lp = xf - mx - jnp.log(z2)

    greedy = jnp.min(jnp.where(kept & (v == mx), topi, jnp.int32(2**30)),
                     axis=1)
    qe = -jnp.log(jnp.clip(topq, 1e-10, 1.0))
    ratio = jnp.where(kept, pr2 / qe, 0.0)
    rmax = jnp.max(ratio, axis=1, keepdims=True)
    rlane = jnp.min(jnp.where(ratio == rmax, j, _K), axis=1, keepdims=True)
    rand_tok = jnp.sum(jnp.where(j == rlane, topi, 0), axis=1)
    sampled = jnp.where(temp[:, 0] < _EPS, greedy, rand_tok).astype(jnp.int32)
    samp_ref[...] = sampled[:, None]

    krank = jax.lax.dot_general(kf, m_lt, (((1,), (0,)), ((), ())),
                                preferred_element_type=jnp.float32,
                            precision=jax.lax.Precision.HIGHEST)
    krank = krank.astype(jnp.int32)

    member = jnp.zeros((b, _K), jnp.float32)
    for c in range(_K):
        hit_c = jnp.max(jnp.where(kept & (topi == c), 1.0, 0.0),
                        axis=1, keepdims=True)
        member = jnp.where(j == c, hit_c, member)
    validc = member == 0.0
    vrank = jax.lax.dot_general(1.0 - member, m_lt, (((1,), (0,)), ((), ())),
                                preferred_element_type=jnp.float32,
                            precision=jax.lax.Precision.HIGHEST)
    vrank = vrank.astype(jnp.int32)

    lane8 = jax.lax.broadcasted_iota(jnp.int32, (b, 8), 1)
    out_lp = jnp.full((b, 8), -jnp.inf, jnp.float32)
    out_idx = jnp.zeros((b, 8), jnp.int32)
    for s in range(8):
        sel = kept & (krank == s)
        lp_s = jnp.max(jnp.where(sel, lp, -jnp.inf), axis=1, keepdims=True)
        idx_s = jnp.sum(jnp.where(sel, topi, 0), axis=1, keepdims=True)
        fsel = validc & (vrank == (s - ln))
        fill_s = jnp.sum(jnp.where(fsel, j, 0), axis=1, keepdims=True)
        pick = s < ln
        out_lp = jnp.where(lane8 == s, jnp.where(pick, lp_s, -jnp.inf),
                           out_lp)
        out_idx = jnp.where(lane8 == s, jnp.where(pick, idx_s, fill_s),
                            out_idx)
    oidx_ref[...] = out_idx
    olp_ref[...] = out_lp


def kernel(logits, temperature, top_k, top_p, q, max_num_logprobs):
    bb, vv = logits.shape
    nb = bb // _RB
    vpad = _NC * 128 - vv
    temp2 = temperature.astype(jnp.float32)[:, None]
    x3 = jnp.pad(logits.astype(jnp.float32), ((0, 0), (0, vpad)),
                 constant_values=_NEG).reshape(bb, _NC, 128)
    q3 = jnp.pad(q.astype(jnp.float32), ((0, 0), (0, vpad)),
                 constant_values=1.0).reshape(bb, _NC, 128)
    topv, topi, topq = pl.pallas_call(
        _topk_extract_kernel,
        grid=(nb,),
        in_specs=[pl.BlockSpec((_RB, _NC, 128), lambda i: (i, 0, 0)),
                  pl.BlockSpec((_RB, _NC, 128), lambda i: (i, 0, 0)),
                  pl.BlockSpec((_RB, 1), lambda i: (i, 0),
                               memory_space=pltpu.SMEM)],
        out_specs=[pl.BlockSpec((_RB, _K, 1), lambda i: (i, 0, 0))] * 3,
        out_shape=[jax.ShapeDtypeStruct((bb, _K, 1), jnp.float32),
                   jax.ShapeDtypeStruct((bb, _K, 1), jnp.int32),
                   jax.ShapeDtypeStruct((bb, _K, 1), jnp.float32)],
        scratch_shapes=[pltpu.VMEM((_RB, _NC, 128), jnp.float32),
                        pltpu.VMEM((_RB, _K, 128), jnp.float32),
                        pltpu.VMEM((_RB, _K, 128), jnp.float32),
                        pltpu.VMEM((_RB, _K, 128), jnp.int32)],
        compiler_params=pltpu.CompilerParams(
            dimension_semantics=("parallel",)),
    )(x3, q3, temp2)
    topv = topv.reshape(bb, _K)
    topi = topi.reshape(bb, _K)
    topq = topq.reshape(bb, _K)

    samp, oidx, olp = pl.pallas_call(
        _finish_kernel,
        out_shape=[jax.ShapeDtypeStruct((bb, 1), jnp.int32),
                   jax.ShapeDtypeStruct((bb, 8), jnp.int32),
                   jax.ShapeDtypeStruct((bb, 8), jnp.float32)],
    )(topv, topi, topq, temp2,
      top_k.astype(jnp.int32)[:, None],
      top_p.astype(jnp.float32)[:, None])
    return samp.reshape(-1), oidx, olp


# speedup vs baseline: 1.2645x; 1.2645x over previous
import jax
import jax.numpy as jnp
from jax.experimental import pallas as pl
from jax.experimental.pallas import tpu as pltpu

_RB = 16
_K = 64
_EPS = 1e-05


_NEG = -1e30
_NC = 784


def _topk_extract_kernel(x_ref, q_ref, t_ref, topv_ref, topi_ref, topq_ref,
                         xs_ref, g_ref, qg_ref, og_ref):
    for row in range(_RB):
        t = t_ref[row, 0]
        t = jnp.where(t < _EPS, 1.0, t)
        xs_ref[row] = x_ref[row] / t

    cm = jnp.max(xs_ref[...], axis=2)

    ncio = jax.lax.broadcasted_iota(jnp.int32, (_RB, _NC), 1)
    lanek = jax.lax.broadcasted_iota(jnp.int32, (_RB, _K), 1)

    def cbody(tt, carry):
        cmx, selc = carry
        m = jnp.max(cmx, axis=1, keepdims=True)
        cid = jnp.min(jnp.where(cmx == m, ncio, _NC), axis=1, keepdims=True)
        cmx = jnp.where(ncio == cid, -jnp.inf, cmx)
        selc = jnp.where(lanek == tt, cid, selc)
        return cmx, selc

    _, selc = jax.lax.fori_loop(
        0, _K, cbody, (cm, jnp.zeros((_RB, _K), jnp.int32)))

    cio2 = jax.lax.broadcasted_iota(jnp.int32, (_K, _NC), 1)
    lane2 = jax.lax.broadcasted_iota(jnp.int32, (_K, 128), 1)
    for row in range(_RB):
        selc_col = jnp.swapaxes(selc[row:row + 1, :], 0, 1)
        oh = (cio2 == selc_col).astype(jnp.float32)
        g_ref[row] = jax.lax.dot_general(
            oh, xs_ref[row], (((1,), (0,)), ((), ())),
            preferred_element_type=jnp.float32,
            precision=jax.lax.Precision.HIGHEST)
        qg_ref[row] = jax.lax.dot_general(
            oh, q_ref[row], (((1,), (0,)), ((), ())),
            preferred_element_type=jnp.float32,
            precision=jax.lax.Precision.HIGHEST)
        og_ref[row] = selc_col * 128 + lane2

    orig3 = og_ref[...]
    qg3 = qg_ref[...]
    lanek3 = jax.lax.broadcasted_iota(jnp.int32, (_RB, _K, 1), 1)

    def ebody(tt, carry):
        gx, topv, topi, topq = carry
        m = jnp.max(jnp.max(gx, axis=2, keepdims=True), axis=1,
                    keepdims=True)
        cand = jnp.where(gx == m, orig3, _NC * 128)
        idx = jnp.min(jnp.min(cand, axis=2, keepdims=True), axis=1,
                      keepdims=True)
        sel = orig3 == idx
        qv = jnp.sum(jnp.sum(jnp.where(sel, qg3, 0.0), axis=2,
                             keepdims=True), axis=1, keepdims=True)
        gx = jnp.where(sel, _NEG, gx)
        hit = lanek3 == tt
        topv = jnp.where(hit, m, topv)
        topi = jnp.where(hit, idx, topi)
        topq = jnp.where(hit, qv, topq)
        return gx, topv, topi, topq

    init = (g_ref[...], jnp.full((_RB, _K, 1), -jnp.inf, jnp.float32),
            jnp.zeros((_RB, _K, 1), jnp.int32),
            jnp.ones((_RB, _K, 1), jnp.float32))
    _, topv, topi, topq = jax.lax.fori_loop(0, _K, ebody, init)
    topv_ref[...] = topv
    topi_ref[...] = topi
    topq_ref[...] = topq


def _finish_kernel(topv_ref, topi_ref, topq_ref, temp_ref, k_ref, p_ref,
                   samp_ref, oidx_ref, olp_ref):
    v = topv_ref[...]
    topi = topi_ref[...]
    topq = topq_ref[...]
    temp = temp_ref[...]
    kk = k_ref[...]
    p = p_ref[...]
    b = v.shape[0]

    j = jax.lax.broadcasted_iota(jnp.int32, (b, _K), 1)
    rio = jax.lax.broadcasted_iota(jnp.int32, (_K, _K), 0)
    cio = jax.lax.broadcasted_iota(jnp.int32, (_K, _K), 1)
    m_lt = (rio < cio).astype(jnp.float32)

    kthv = jnp.max(jnp.where(j == jnp.maximum(kk, 1) - 1, v, -jnp.inf),
                   axis=1, keepdims=True)
    keep_k = v >= kthv
    mx = v[:, 0:1]
    e = jnp.exp(jnp.where(keep_k, v, -jnp.inf) - mx)
    z = jnp.sum(e, axis=1, keepdims=True)
    pr = e / z

    ntie_before = jnp.zeros((b, _K), jnp.float32)
    ntie_after = jnp.zeros((b, _K), jnp.float32)
    run = jnp.zeros((b, 1), jnp.float32)
    prev = jnp.full((b, 1), jnp.inf, jnp.float32)
    for t in range(_K):
        v_t = jnp.max(jnp.where(j == t, v, -jnp.inf), axis=1, keepdims=True)
        run = jnp.where(v_t == prev, run + 1.0, 0.0)
        ntie_before = jnp.where(j == t, run, ntie_before)
        prev = v_t
    run = jnp.zeros((b, 1), jnp.float32)
    nxt = jnp.full((b, 1), jnp.inf, jnp.float32)
    for t in range(_K - 1, -1, -1):
        v_t = jnp.max(jnp.where(j == t, v, -jnp.inf), axis=1, keepdims=True)
        run = jnp.where(v_t == nxt, run + 1.0, 0.0)
        ntie_after = jnp.where(j == t, run, ntie_after)
        nxt = v_t

    s_inc = jax.lax.dot_general(pr, (rio <= cio).astype(jnp.float32),
                                (((1,), (0,)), ((), ())),
                                preferred_element_type=jnp.float32,
                            precision=jax.lax.Precision.HIGHEST)
    sgt = (s_inc - pr) - pr * ntie_before
    s_prev = sgt + pr * ntie_after
    is_top0 = (v == mx) & (ntie_after == 0.0)
    keep_p = is_top0 | (s_prev < p)
    kept = keep_k & keep_p
    kf = kept.astype(jnp.float32)
    lnf = jnp.sum(kf, axis=1, keepdims=True)
    ln = lnf.astype(jnp.int32)

    xf = jnp.where(kept, v, -jnp.inf)
    e2 = jnp.exp(xf - mx)
    z2 = jnp.sum(e2, axis=1, keepdims=True)
    pr2 = e2 / z2
    lp = xf - mx - jnp.log(z2)

    greedy = jnp.min(jnp.where(kept & (v == mx), topi, jnp.int32(2**30)),
                     axis=1)
    qe = -jnp.log(jnp.clip(topq, 1e-10, 1.0))
    ratio = jnp.where(kept, pr2 / qe, 0.0)
    rmax = jnp.max(ratio, axis=1, keepdims=True)
    rlane = jnp.min(jnp.where(ratio == rmax, j, _K), axis=1, keepdims=True)
    rand_tok = jnp.sum(jnp.where(j == rlane, topi, 0), axis=1)
    sampled = jnp.where(temp[:, 0] < _EPS, greedy, rand_tok).astype(jnp.int32)
    samp_ref[...] = sampled[:, None]

    krank = jax.lax.dot_general(kf, m_lt, (((1,), (0,)), ((), ())),
                                preferred_element_type=jnp.float32,
                            precision=jax.lax.Precision.HIGHEST)
    krank = krank.astype(jnp.int32)

    member = jnp.zeros((b, _K), jnp.float32)
    for c in range(_K):
        hit_c = jnp.max(jnp.where(kept & (topi == c), 1.0, 0.0),
                        axis=1, keepdims=True)
        member = jnp.where(j == c, hit_c, member)
    validc = member == 0.0
    vrank = jax.lax.dot_general(1.0 - member, m_lt, (((1,), (0,)), ((), ())),
                                preferred_element_type=jnp.float32,
                            precision=jax.lax.Precision.HIGHEST)
    vrank = vrank.astype(jnp.int32)

    lane8 = jax.lax.broadcasted_iota(jnp.int32, (b, 8), 1)
    out_lp = jnp.full((b, 8), -jnp.inf, jnp.float32)
    out_idx = jnp.zeros((b, 8), jnp.int32)
    for s in range(8):
        sel = kept & (krank == s)
        lp_s = jnp.max(jnp.where(sel, lp, -jnp.inf), axis=1, keepdims=True)
        idx_s = jnp.sum(jnp.where(sel, topi, 0), axis=1, keepdims=True)
        fsel = validc & (vrank == (s - ln))
        fill_s = jnp.sum(jnp.where(fsel, j, 0), axis=1, keepdims=True)
        pick = s < ln
        out_lp = jnp.where(lane8 == s, jnp.where(pick, lp_s, -jnp.inf),
                           out_lp)
        out_idx = jnp.where(lane8 == s, jnp.where(pick, idx_s, fill_s),
                            out_idx)
    oidx_ref[...] = out_idx
    olp_ref[...] = out_lp


def kernel(logits, temperature, top_k, top_p, q, max_num_logprobs):
    bb, vv = logits.shape
    nb = bb // _RB
    vpad = _NC * 128 - vv
    temp2 = temperature.astype(jnp.float32)[:, None]
    x3 = jnp.pad(logits.astype(jnp.float32), ((0, 0), (0, vpad)),
                 constant_values=_NEG).reshape(bb, _NC, 128)
    q3 = jnp.pad(q.astype(jnp.float32), ((0, 0), (0, vpad)),
                 constant_values=1.0).reshape(bb, _NC, 128)
    topv, topi, topq = pl.pallas_call(
        _topk_extract_kernel,
        grid=(nb,),
        in_specs=[pl.BlockSpec((_RB, _NC, 128), lambda i: (i, 0, 0)),
                  pl.BlockSpec((_RB, _NC, 128), lambda i: (i, 0, 0)),
                  pl.BlockSpec((_RB, 1), lambda i: (i, 0),
                               memory_space=pltpu.SMEM)],
        out_specs=[pl.BlockSpec((_RB, _K, 1), lambda i: (i, 0, 0))] * 3,
        out_shape=[jax.ShapeDtypeStruct((bb, _K, 1), jnp.float32),
                   jax.ShapeDtypeStruct((bb, _K, 1), jnp.int32),
                   jax.ShapeDtypeStruct((bb, _K, 1), jnp.float32)],
        scratch_shapes=[pltpu.VMEM((_RB, _NC, 128), jnp.float32),
                        pltpu.VMEM((_RB, _K, 128), jnp.float32),
                        pltpu.VMEM((_RB, _K, 128), jnp.float32),
                        pltpu.VMEM((_RB, _K, 128), jnp.int32)],
        compiler_params=pltpu.CompilerParams(
            dimension_semantics=("parallel",)),
    )(x3, q3, temp2)
    topv = topv.reshape(bb, _K)
    topi = topi.reshape(bb, _K)
    topq = topq.reshape(bb, _K)

    samp, oidx, olp = pl.pallas_call(
        _finish_kernel,
        out_shape=[jax.ShapeDtypeStruct((bb, 1), jnp.int32),
                   jax.ShapeDtypeStruct((bb, 8), jnp.int32),
                   jax.ShapeDtypeStruct((bb, 8), jnp.float32)],
    )(topv, topi, topq, temp2,
      top_k.astype(jnp.int32)[:, None],
      top_p.astype(jnp.float32)[:, None])
    return samp.reshape(-1), oidx, olp


# speedup vs baseline: 1.3889x; 1.0984x over previous
import jax
import jax.numpy as jnp
from jax.experimental import pallas as pl
from jax.experimental.pallas import tpu as pltpu

_RB = 16
_K = 64
_EPS = 1e-05
_KE = 56


_NEG = -1e30
_NC = 784


def _topk_extract_kernel(x_ref, q_ref, t_ref, topv_ref, topi_ref, topq_ref,
                         xs_ref, g_ref, qg_ref, og_ref):
    for row in range(_RB):
        t = t_ref[row, 0]
        t = jnp.where(t < _EPS, 1.0, t)
        xs_ref[row] = x_ref[row] / t

    cm = jnp.max(xs_ref[...], axis=2)

    ncio = jax.lax.broadcasted_iota(jnp.int32, (_RB, _NC), 1)
    lanek = jax.lax.broadcasted_iota(jnp.int32, (_RB, _K), 1)

    def cbody(tt, carry):
        cmx, selc = carry
        m = jnp.max(cmx, axis=1, keepdims=True)
        cid = jnp.min(jnp.where(cmx == m, ncio, _NC), axis=1, keepdims=True)
        cmx = jnp.where(ncio == cid, -jnp.inf, cmx)
        selc = jnp.where(lanek == tt, cid, selc)
        return cmx, selc

    _, selc = jax.lax.fori_loop(
        0, _KE, cbody, (cm, jnp.zeros((_RB, _K), jnp.int32)))

    cio2 = jax.lax.broadcasted_iota(jnp.int32, (_K, _NC), 1)
    lane2 = jax.lax.broadcasted_iota(jnp.int32, (_K, 128), 1)
    for row in range(_RB):
        selc_col = jnp.swapaxes(selc[row:row + 1, :], 0, 1)
        oh = (cio2 == selc_col).astype(jnp.float32)
        g_ref[row] = jax.lax.dot_general(
            oh, xs_ref[row], (((1,), (0,)), ((), ())),
            preferred_element_type=jnp.float32,
            precision=jax.lax.Precision.HIGHEST)
        qg_ref[row] = jax.lax.dot_general(
            oh, q_ref[row], (((1,), (0,)), ((), ())),
            preferred_element_type=jnp.float32,
            precision=jax.lax.Precision.HIGHEST)
        og_ref[row] = selc_col * 128 + lane2

    orig3 = og_ref[...]
    qg3 = qg_ref[...]
    lanek3 = jax.lax.broadcasted_iota(jnp.int32, (_RB, _K, 1), 1)

    def ebody(tt, carry):
        gx, topv, topi, topq = carry
        m = jnp.max(jnp.max(gx, axis=2, keepdims=True), axis=1,
                    keepdims=True)
        cand = jnp.where(gx == m, orig3, _NC * 128)
        idx = jnp.min(jnp.min(cand, axis=2, keepdims=True), axis=1,
                      keepdims=True)
        sel = orig3 == idx
        qv = jnp.sum(jnp.sum(jnp.where(sel, qg3, 0.0), axis=2,
                             keepdims=True), axis=1, keepdims=True)
        gx = jnp.where(sel, _NEG, gx)
        hit = lanek3 == tt
        topv = jnp.where(hit, m, topv)
        topi = jnp.where(hit, idx, topi)
        topq = jnp.where(hit, qv, topq)
        return gx, topv, topi, topq

    init = (g_ref[...], jnp.full((_RB, _K, 1), -jnp.inf, jnp.float32),
            jnp.zeros((_RB, _K, 1), jnp.int32),
            jnp.ones((_RB, _K, 1), jnp.float32))
    _, topv, topi, topq = jax.lax.fori_loop(0, _KE, ebody, init)
    topv_ref[...] = topv
    topi_ref[...] = topi
    topq_ref[...] = topq


def _finish_kernel(topv_ref, topi_ref, topq_ref, temp_ref, k_ref, p_ref,
                   samp_ref, oidx_ref, olp_ref):
    v = topv_ref[...]
    topi = topi_ref[...]
    topq = topq_ref[...]
    temp = temp_ref[...]
    kk = k_ref[...]
    p = p_ref[...]
    b = v.shape[0]

    j = jax.lax.broadcasted_iota(jnp.int32, (b, _K), 1)
    rio = jax.lax.broadcasted_iota(jnp.int32, (_K, _K), 0)
    cio = jax.lax.broadcasted_iota(jnp.int32, (_K, _K), 1)
    m_lt = (rio < cio).astype(jnp.float32)

    kthv = jnp.max(jnp.where(j == jnp.maximum(kk, 1) - 1, v, -jnp.inf),
                   axis=1, keepdims=True)
    keep_k = v >= kthv
    mx = v[:, 0:1]
    e = jnp.exp(jnp.where(keep_k, v, -jnp.inf) - mx)
    z = jnp.sum(e, axis=1, keepdims=True)
    pr = e / z

    ntie_before = jnp.zeros((b, _K), jnp.float32)
    ntie_after = jnp.zeros((b, _K), jnp.float32)
    run = jnp.zeros((b, 1), jnp.float32)
    prev = jnp.full((b, 1), jnp.inf, jnp.float32)
    for t in range(_K):
        v_t = jnp.max(jnp.where(j == t, v, -jnp.inf), axis=1, keepdims=True)
        run = jnp.where(v_t == prev, run + 1.0, 0.0)
        ntie_before = jnp.where(j == t, run, ntie_before)
        prev = v_t
    run = jnp.zeros((b, 1), jnp.float32)
    nxt = jnp.full((b, 1), jnp.inf, jnp.float32)
    for t in range(_K - 1, -1, -1):
        v_t = jnp.max(jnp.where(j == t, v, -jnp.inf), axis=1, keepdims=True)
        run = jnp.where(v_t == nxt, run + 1.0, 0.0)
        ntie_after = jnp.where(j == t, run, ntie_after)
        nxt = v_t

    s_inc = jax.lax.dot_general(pr, (rio <= cio).astype(jnp.float32),
                                (((1,), (0,)), ((), ())),
                                preferred_element_type=jnp.float32,
                            precision=jax.lax.Precision.HIGHEST)
    sgt = (s_inc - pr) - pr * ntie_before
    s_prev = sgt + pr * ntie_after
    is_top0 = (v == mx) & (ntie_after == 0.0)
    keep_p = is_top0 | (s_prev < p)
    kept = keep_k & keep_p
    kf = kept.astype(jnp.float32)
    lnf = jnp.sum(kf, axis=1, keepdims=True)
    ln = lnf.astype(jnp.int32)

    xf = jnp.where(kept, v, -jnp.inf)
    e2 = jnp.exp(xf - mx)
    z2 = jnp.sum(e2, axis=1, keepdims=True)
    pr2 = e2 / z2
    lp = xf - mx - jnp.log(z2)

    greedy = jnp.min(jnp.where(kept & (v == mx), topi, jnp.int32(2**30)),
                     axis=1)
    qe = -jnp.log(jnp.clip(topq, 1e-10, 1.0))
    ratio = jnp.where(kept, pr2 / qe, 0.0)
    rmax = jnp.max(ratio, axis=1, keepdims=True)
    rlane = jnp.min(jnp.where(ratio == rmax, j, _K), axis=1, keepdims=True)
    rand_tok = jnp.sum(jnp.where(j == rlane, topi, 0), axis=1)
    sampled = jnp.where(temp[:, 0] < _EPS, greedy, rand_tok).astype(jnp.int32)
    samp_ref[...] = sampled[:, None]

    krank = jax.lax.dot_general(kf, m_lt, (((1,), (0,)), ((), ())),
                                preferred_element_type=jnp.float32,
                            precision=jax.lax.Precision.HIGHEST)
    krank = krank.astype(jnp.int32)

    member = jnp.zeros((b, _K), jnp.float32)
    for c in range(_K):
        hit_c = jnp.max(jnp.where(kept & (topi == c), 1.0, 0.0),
                        axis=1, keepdims=True)
        member = jnp.where(j == c, hit_c, member)
    validc = member == 0.0
    vrank = jax.lax.dot_general(1.0 - member, m_lt, (((1,), (0,)), ((), ())),
                                preferred_element_type=jnp.float32,
                            precision=jax.lax.Precision.HIGHEST)
    vrank = vrank.astype(jnp.int32)

    lane8 = jax.lax.broadcasted_iota(jnp.int32, (b, 8), 1)
    out_lp = jnp.full((b, 8), -jnp.inf, jnp.float32)
    out_idx = jnp.zeros((b, 8), jnp.int32)
    for s in range(8):
        sel = kept & (krank == s)
        lp_s = jnp.max(jnp.where(sel, lp, -jnp.inf), axis=1, keepdims=True)
        idx_s = jnp.sum(jnp.where(sel, topi, 0), axis=1, keepdims=True)
        fsel = validc & (vrank == (s - ln))
        fill_s = jnp.sum(jnp.where(fsel, j, 0), axis=1, keepdims=True)
        pick = s < ln
        out_lp = jnp.where(lane8 == s, jnp.where(pick, lp_s, -jnp.inf),
                           out_lp)
        out_idx = jnp.where(lane8 == s, jnp.where(pick, idx_s, fill_s),
                            out_idx)
    oidx_ref[...] = out_idx
    olp_ref[...] = out_lp


def kernel(logits, temperature, top_k, top_p, q, max_num_logprobs):
    bb, vv = logits.shape
    nb = bb // _RB
    vpad = _NC * 128 - vv
    temp2 = temperature.astype(jnp.float32)[:, None]
    x3 = jnp.pad(logits.astype(jnp.float32), ((0, 0), (0, vpad)),
                 constant_values=_NEG).reshape(bb, _NC, 128)
    q3 = jnp.pad(q.astype(jnp.float32), ((0, 0), (0, vpad)),
                 constant_values=1.0).reshape(bb, _NC, 128)
    topv, topi, topq = pl.pallas_call(
        _topk_extract_kernel,
        grid=(nb,),
        in_specs=[pl.BlockSpec((_RB, _NC, 128), lambda i: (i, 0, 0)),
                  pl.BlockSpec((_RB, _NC, 128), lambda i: (i, 0, 0)),
                  pl.BlockSpec((_RB, 1), lambda i: (i, 0),
                               memory_space=pltpu.SMEM)],
        out_specs=[pl.BlockSpec((_RB, _K, 1), lambda i: (i, 0, 0))] * 3,
        out_shape=[jax.ShapeDtypeStruct((bb, _K, 1), jnp.float32),
                   jax.ShapeDtypeStruct((bb, _K, 1), jnp.int32),
                   jax.ShapeDtypeStruct((bb, _K, 1), jnp.float32)],
        scratch_shapes=[pltpu.VMEM((_RB, _NC, 128), jnp.float32),
                        pltpu.VMEM((_RB, _K, 128), jnp.float32),
                        pltpu.VMEM((_RB, _K, 128), jnp.float32),
                        pltpu.VMEM((_RB, _K, 128), jnp.int32)],
        compiler_params=pltpu.CompilerParams(
            dimension_semantics=("parallel",)),
    )(x3, q3, temp2)
    topv = topv.reshape(bb, _K)
    topi = topi.reshape(bb, _K)
    topq = topq.reshape(bb, _K)

    samp, oidx, olp = pl.pallas_call(
        _finish_kernel,
        out_shape=[jax.ShapeDtypeStruct((bb, 1), jnp.int32),
                   jax.ShapeDtypeStruct((bb, 8), jnp.int32),
                   jax.ShapeDtypeStruct((bb, 8), jnp.float32)],
    )(topv, topi, topq, temp2,
      top_k.astype(jnp.int32)[:, None],
      top_p.astype(jnp.float32)[:, None])
    return samp.reshape(-1), oidx, olp
